# baseline (speedup 1.0000x reference)
import functools

import jax
import jax.numpy as jnp
from jax import lax
from jax.experimental import pallas as pl
from jax.experimental.pallas import tpu as pltpu
from jax.experimental.pallas import tpu_sc as plsc

EMBED = 32
LANES = 16
WAVE = 2
PHASES = 4
SUBWAVES = LANES // WAVE
AHEAD = PHASES - 1


@functools.lru_cache(maxsize=None)
def _make_sc_lookup(batch: int):
    info = plsc.get_sparse_core_info()
    nw = info.num_cores * info.num_subcores
    bw = batch // nw
    assert batch % nw == 0 and bw % LANES == 0
    nchunks = bw // LANES
    assert SUBWAVES % PHASES == 0
    mesh = plsc.VectorSubcoreMesh(core_axis_name="c", subcore_axis_name="s")

    @functools.partial(
        pl.kernel,
        mesh=mesh,
        compiler_params=pltpu.CompilerParams(needs_layout_passes=False),
        out_type=jax.ShapeDtypeStruct((2 * EMBED, batch), jnp.float32),
        scratch_types=[
            pltpu.VMEM((bw,), jnp.int32),
            pltpu.VMEM((bw,), jnp.int32),
            pltpu.VMEM((PHASES, WAVE, EMBED, 128), jnp.float32),
            pltpu.VMEM((PHASES, WAVE, EMBED, 128), jnp.float32),
            pltpu.VMEM((EMBED, bw), jnp.float32),
            pltpu.VMEM((EMBED, bw), jnp.float32),
        ] + [pltpu.SemaphoreType.DMA] * (2 * PHASES + 1),
    )
    def sc_lookup(idx_u_hbm, idx_l_hbm, emb_u_hbm, emb_l_hbm, out_hbm,
                  idx_u_v, idx_l_v, buf_u, buf_l, rows_u, rows_l, *sems):
        sems_u = sems[:PHASES]
        sems_l = sems[PHASES:2 * PHASES]
        sem_w = sems[2 * PHASES]
        wid = lax.axis_index("s") * info.num_cores + lax.axis_index("c")
        base = wid * bw
        pltpu.sync_copy(idx_u_hbm.at[pl.ds(base, bw)], idx_u_v)
        pltpu.sync_copy(idx_l_hbm.at[pl.ds(base, bw)], idx_l_v)

        c_lo = lax.iota(jnp.int32, LANES)
        c_hi = c_lo + LANES

        def fire(iu_vec, il_vec, sw, ph):
            for j in range(WAVE):
                k = sw * WAVE + j
                cu = pl.multiple_of((iu_vec[k] >> 7) * 128, 128)
                cl = pl.multiple_of((il_vec[k] >> 7) * 128, 128)
                pltpu.async_copy(
                    emb_u_hbm.at[:, pl.ds(cu, 128)], buf_u.at[ph, j],
                    sems_u[ph])
                pltpu.async_copy(
                    emb_l_hbm.at[:, pl.ds(cl, 128)], buf_l.at[ph, j],
                    sems_l[ph])

        def drain_extract(iu_vec, il_vec, u0, sw, ph):
            for j in range(WAVE):
                pltpu.make_async_copy(
                    emb_u_hbm.at[:, pl.ds(0, 128)], buf_u.at[ph, j],
                    sems_u[ph]).wait()
                pltpu.make_async_copy(
                    emb_l_hbm.at[:, pl.ds(0, 128)], buf_l.at[ph, j],
                    sems_l[ph]).wait()
            for j in range(WAVE):
                k = sw * WAVE + j
                lu = jnp.broadcast_to(iu_vec[k] & 127, (LANES,))
                ll = jnp.broadcast_to(il_vec[k] & 127, (LANES,))
                us = jnp.broadcast_to(u0 + k, (LANES,))
                v0 = plsc.load_gather(buf_u.at[ph, j], [c_lo, lu])
                v1 = plsc.load_gather(buf_u.at[ph, j], [c_hi, lu])
                plsc.store_scatter(rows_u, [c_lo, us], v0)
                plsc.store_scatter(rows_u, [c_hi, us], v1)
                w0 = plsc.load_gather(buf_l.at[ph, j], [c_lo, ll])
                w1 = plsc.load_gather(buf_l.at[ph, j], [c_hi, ll])
                plsc.store_scatter(rows_l, [c_lo, us], w0)
                plsc.store_scatter(rows_l, [c_hi, us], w1)

        iu0 = idx_u_v[pl.ds(0, LANES)]
        il0 = idx_l_v[pl.ds(0, LANES)]
        for g in range(AHEAD):
            fire(iu0, il0, g, g % PHASES)

        def do_chunk(c, carry):
            u0 = c * LANES
            iu_vec = idx_u_v[pl.ds(u0, LANES)]
            il_vec = idx_l_v[pl.ds(u0, LANES)]
            for sw in range(SUBWAVES):
                n = sw + AHEAD
                ph_fire = n % PHASES
                if n < SUBWAVES:
                    fire(iu_vec, il_vec, n, ph_fire)
                else:
                    @pl.when(c + 1 < nchunks)
                    def _():
                        iun = idx_u_v[pl.ds((c + 1) * LANES, LANES)]
                        iln = idx_l_v[pl.ds((c + 1) * LANES, LANES)]
                        fire(iun, iln, n - SUBWAVES, ph_fire)
                drain_extract(iu_vec, il_vec, u0, sw, sw % PHASES)
            return carry

        lax.fori_loop(0, nchunks, do_chunk, 0)

        pltpu.async_copy(
            rows_u, out_hbm.at[pl.ds(0, EMBED), pl.ds(base, bw)], sem_w
        ).wait()
        pltpu.async_copy(
            rows_l, out_hbm.at[pl.ds(EMBED, EMBED), pl.ds(base, bw)], sem_w
        ).wait()

    return sc_lookup


TC_STEP = 8


@functools.lru_cache(maxsize=None)
def _make_tc_lookup(n_tc: int):
    steps = n_tc // TC_STEP
    assert n_tc % TC_STEP == 0

    group = 128 // TC_STEP

    def body(iu_ref, il_ref, *refs):
        blocks = refs[:-1]
        out_ref = refs[-1]
        i = pl.program_id(0)
        lane = lax.broadcasted_iota(jnp.int32, (EMBED, 128), 1)
        acc_u = out_ref[pl.ds(0, EMBED), :]
        acc_l = out_ref[pl.ds(EMBED, EMBED), :]
        for j in range(TC_STEP):
            col = (i % group) * TC_STEP + j
            lu = iu_ref[i * TC_STEP + j] & 127
            ll = il_ref[i * TC_STEP + j] & 127
            ru = pltpu.roll(blocks[j][...], (col - lu) & 127, axis=1)
            rl = pltpu.roll(blocks[TC_STEP + j][...], (col - ll) & 127,
                            axis=1)
            sel = lane == col
            acc_u = jnp.where(sel, ru, acc_u)
            acc_l = jnp.where(sel, rl, acc_l)
        out_ref[pl.ds(0, EMBED), :] = acc_u
        out_ref[pl.ds(EMBED, EMBED), :] = acc_l

    def u_map(j):
        return lambda i, iu, il: (0, iu[i * TC_STEP + j] >> 7)

    def l_map(j):
        return lambda i, iu, il: (0, il[i * TC_STEP + j] >> 7)

    in_specs = (
        [pl.BlockSpec((EMBED, 128), u_map(j)) for j in range(TC_STEP)]
        + [pl.BlockSpec((EMBED, 128), l_map(j)) for j in range(TC_STEP)]
    )
    grid_spec = pltpu.PrefetchScalarGridSpec(
        num_scalar_prefetch=2,
        grid=(steps,),
        in_specs=in_specs,
        out_specs=pl.BlockSpec((2 * EMBED, 128),
                               lambda i, iu, il: (0, i // group)),
    )

    def tc_lookup(idx_u, idx_l, emb_u_t, emb_l_t):
        ins = [emb_u_t] * TC_STEP + [emb_l_t] * TC_STEP
        return pl.pallas_call(
            body,
            grid_spec=grid_spec,
            out_shape=jax.ShapeDtypeStruct((2 * EMBED, n_tc), jnp.float32),
        )(idx_u, idx_l, *ins)

    return tc_lookup


TC_FRACTION_NUM = 4096


def kernel(user_fea, emb_user, emb_location):
    batch = user_fea.shape[0]
    idx_u = user_fea[:, 0].astype(jnp.int32)
    idx_l = user_fea[:, 1].astype(jnp.int32)
    emb_u_t = emb_user.T
    emb_l_t = emb_location.T
    n_tc = TC_FRACTION_NUM if batch > TC_FRACTION_NUM else 0
    n_sc = batch - n_tc
    if n_tc == 0 or n_sc % 4096 != 0:
        out_t = _make_sc_lookup(batch)(idx_u, idx_l, emb_u_t, emb_l_t)
        return out_t.T
    sc_t = _make_sc_lookup(n_sc)(
        idx_u[:n_sc], idx_l[:n_sc], emb_u_t, emb_l_t)
    tc_t = _make_tc_lookup(n_tc)(
        idx_u[n_sc:], idx_l[n_sc:], emb_u_t, emb_l_t)
    return jnp.concatenate([sc_t, tc_t], axis=1).T
